# Initial kernel scaffold; baseline (speedup 1.0000x reference)
import functools

import jax
import jax.numpy as jnp
from jax import lax
from jax.experimental import pallas as pl
from jax.experimental.pallas import tpu as pltpu
from jax.experimental.pallas import tpu_sc as plsc

GENE_NUM = 100000
D = 64
BATCH = 4096
SEQ = 200
EPS = 1e-5

NC = 2
NS = 16
NW = NC * NS
NTOT = BATCH * SEQ
PER_W = NTOT // NW
CW = 128
NCH = PER_W // CW
NB = 4


def _ln_body(t_ref, g_ref, b_ref, o_ref):
    t = t_ref[...]
    m = jnp.mean(t, axis=-1, keepdims=True)
    d = t - m
    v = jnp.mean(d * d, axis=-1, keepdims=True)
    o_ref[...] = (d / jnp.sqrt(v + EPS)) * g_ref[...] + b_ref[...]


def _normalize_table(table, gamma, beta):
    rows_blk = GENE_NUM // 10
    return pl.pallas_call(
        _ln_body,
        grid=(GENE_NUM // rows_blk,),
        in_specs=[
            pl.BlockSpec((rows_blk, D), lambda i: (i, 0)),
            pl.BlockSpec((1, D), lambda i: (0, 0)),
            pl.BlockSpec((1, D), lambda i: (0, 0)),
        ],
        out_specs=pl.BlockSpec((rows_blk, D), lambda i: (i, 0)),
        out_shape=jax.ShapeDtypeStruct((GENE_NUM, D), jnp.float32),
    )(table, gamma, beta)


@functools.partial(
    pl.kernel,
    mesh=plsc.VectorSubcoreMesh(core_axis_name="c", subcore_axis_name="s"),
    out_type=jax.ShapeDtypeStruct((NTOT, D), jnp.float32),
    scratch_types=[
        pltpu.VMEM((NCH, CW), jnp.int32),
        pltpu.VMEM((NB, CW, D), jnp.float32),
        pltpu.SemaphoreType.DMA,
        pltpu.SemaphoreType.DMA,
    ],
)
def _gather_kernel(table_hbm, idx_hbm, out_hbm, idx_v, rows_v, gsem, osem):
    wid = lax.axis_index("s") * NC + lax.axis_index("c")
    base = wid * PER_W

    pltpu.sync_copy(idx_hbm.at[wid], idx_v)

    for s in range(NB - 1):
        pltpu.async_copy(table_hbm.at[idx_v.at[s]], rows_v.at[s], gsem)

    def chunk_step(g, slot):
        pltpu.make_async_copy(
            table_hbm.at[idx_v.at[g]], rows_v.at[slot], gsem).wait()
        out_slice = out_hbm.at[pl.ds(base + g * CW, CW)]
        pltpu.async_copy(rows_v.at[slot], out_slice, osem)

        @pl.when(g > 0)
        def _():
            pltpu.make_async_copy(
                rows_v.at[slot], out_slice, osem).wait()

        @pl.when(g + NB - 1 < NCH)
        def _():
            nxt = g + NB - 1
            prev_slot = (slot + NB - 1) % NB
            pltpu.async_copy(
                table_hbm.at[idx_v.at[nxt]], rows_v.at[prev_slot], gsem)

    def outer(t, _):
        for b in range(NB):
            chunk_step(t * NB + b, b)
        return 0

    lax.fori_loop(0, NCH // NB, outer, 0)

    pltpu.make_async_copy(
        rows_v.at[0], out_hbm.at[pl.ds(base, CW)], osem).wait()


def kernel(x, table, gamma, beta):
    ntab = _normalize_table(table, gamma.reshape(1, D), beta.reshape(1, D))
    x3 = x.astype(jnp.int32).reshape(NW, NCH, CW)
    out = _gather_kernel(ntab, x3)
    return out.reshape(BATCH, SEQ, D)


# speedup vs baseline: 4.0761x; 4.0761x over previous
import functools

import jax
import jax.numpy as jnp
from jax import lax
from jax.experimental import pallas as pl
from jax.experimental.pallas import tpu as pltpu
from jax.experimental.pallas import tpu_sc as plsc

GENE_NUM = 100000
D = 64
BATCH = 4096
SEQ = 200
EPS = 1e-5

NC = 2
NS = 16
NW = NC * NS
NTOT = BATCH * SEQ
PER_W = NTOT // NW
CW = 128
NCH = PER_W // CW
NB = 4
NG = NCH // NB


def _ln_body(t_ref, g_ref, b_ref, o_ref):
    t = t_ref[...]
    m = jnp.mean(t, axis=-1, keepdims=True)
    d = t - m
    v = jnp.mean(d * d, axis=-1, keepdims=True)
    o_ref[...] = (d / jnp.sqrt(v + EPS)) * g_ref[...] + b_ref[...]


def _normalize_table(table, gamma, beta):
    rows_blk = GENE_NUM // 10
    return pl.pallas_call(
        _ln_body,
        grid=(GENE_NUM // rows_blk,),
        in_specs=[
            pl.BlockSpec((rows_blk, D), lambda i: (i, 0)),
            pl.BlockSpec((1, D), lambda i: (0, 0)),
            pl.BlockSpec((1, D), lambda i: (0, 0)),
        ],
        out_specs=pl.BlockSpec((rows_blk, D), lambda i: (i, 0)),
        out_shape=jax.ShapeDtypeStruct((GENE_NUM, D), jnp.float32),
    )(table, gamma, beta)


@functools.partial(
    pl.kernel,
    mesh=plsc.VectorSubcoreMesh(core_axis_name="c", subcore_axis_name="s"),
    compiler_params=pltpu.CompilerParams(use_tc_tiling_on_sc=False),
    out_type=jax.ShapeDtypeStruct((NTOT, D), jnp.float32),
    scratch_types=[
        pltpu.VMEM((NCH, CW), jnp.int32),
        pltpu.VMEM((2 * NB, CW, D), jnp.float32),
        pltpu.SemaphoreType.DMA,
        pltpu.SemaphoreType.DMA,
    ],
)
def _gather_kernel(table_hbm, idx_hbm, out_hbm, idx_v, rows_v, gsem, osem):
    wid = lax.axis_index("s") * NC + lax.axis_index("c")
    base = wid * PER_W

    pltpu.sync_copy(idx_hbm.at[wid], idx_v)

    def gather_chunk(c, slot):
        return pltpu.async_copy(
            table_hbm.at[idx_v.at[c]], rows_v.at[slot], gsem)

    def drain(sem):
        pltpu.make_async_copy(
            rows_v.at[0], out_hbm.at[pl.ds(base, CW)], sem).wait()

    for b in range(NB):
        gather_chunk(b, b)

    def group_step(t, t2, par):
        off = par * NB
        if par == 1:
            for _ in range(NB):
                drain(osem)
        else:
            @pl.when(t2 > 0)
            def _():
                for _ in range(NB):
                    drain(osem)
        for _ in range(NB):
            drain(gsem)
        if par == 0:
            for b in range(NB):
                gather_chunk((t + 1) * NB + b, NB + b)
        else:
            @pl.when(t2 < NG // 2 - 1)
            def _():
                for b in range(NB):
                    gather_chunk((t + 1) * NB + b, b)
        for b in range(NB):
            c = t * NB + b
            pltpu.async_copy(
                rows_v.at[off + b],
                out_hbm.at[pl.ds(base + c * CW, CW)], osem)

    def outer(t2, _):
        group_step(2 * t2, t2, 0)
        group_step(2 * t2 + 1, t2, 1)
        return 0

    lax.fori_loop(0, NG // 2, outer, 0)

    for _ in range(NB):
        drain(osem)


def kernel(x, table, gamma, beta):
    ntab = _normalize_table(table, gamma.reshape(1, D), beta.reshape(1, D))
    x3 = x.astype(jnp.int32).reshape(NW, NCH, CW)
    out = _gather_kernel(ntab, x3)
    return out.reshape(BATCH, SEQ, D)
